# Initial kernel scaffold; baseline (speedup 1.0000x reference)
import jax
import jax.numpy as jnp
from jax.experimental import pallas as pl


def kernel(word_ids, topic_ids, word_table, topic_table, W_word, b_word, W_topic, b_topic, gamma, beta):
    raise NotImplementedError("write your pallas kernel here")


# speedup vs baseline: 1.8545x; 1.8545x over previous
import functools

import jax
import jax.numpy as jnp
from jax import lax
from jax.experimental import pallas as pl
from jax.experimental.pallas import tpu as pltpu
from jax.experimental.pallas import tpu_sc as plsc

V, DW, T, DT, H = 100000, 128, 512, 64, 256
B, L = 4096, 50
N_ROWS = B * L

_NC, _NS = 2, 16
_NW = _NC * _NS
_CHUNK = 128
_ROWS_PER_W = N_ROWS // _NW
_CHUNKS_PER_W = _ROWS_PER_W // _CHUNK


def _sc_gather_body(ids_hbm, table_hbm, out_hbm, idx_all, rows, sem):
    wid = lax.axis_index("s") * _NC + lax.axis_index("c")
    chunk_base = wid * _CHUNKS_PER_W
    pltpu.sync_copy(ids_hbm.at[wid], idx_all)

    def chunk(j, carry):
        pltpu.async_copy(table_hbm.at[idx_all.at[j]], rows, sem).wait()
        pltpu.sync_copy(rows, out_hbm.at[pl.ds((chunk_base + j) * _CHUNK, _CHUNK)])
        return carry

    lax.fori_loop(0, _CHUNKS_PER_W, chunk, 0)


def _sc_gather(word_ids_flat, word_table):
    ids2d = word_ids_flat.reshape(_NW, _CHUNKS_PER_W, _CHUNK)
    mesh = plsc.VectorSubcoreMesh(core_axis_name="c", subcore_axis_name="s")
    k = functools.partial(
        pl.kernel,
        mesh=mesh,
        out_type=jax.ShapeDtypeStruct((N_ROWS, DW), jnp.float32),
        scratch_types=[
            pltpu.VMEM((_CHUNKS_PER_W, _CHUNK), jnp.int32),
            pltpu.VMEM((_CHUNK, DW), jnp.float32),
            pltpu.SemaphoreType.DMA,
        ],
    )(_sc_gather_body)
    return k(ids2d, word_table)


def _tc_body(we_ref, wid_ref, tid_ref, tt_ref, wtt_ref, wwt_ref,
             bw_ref, bt_ref, g_ref, b_ref, out_ref):
    bb = tid_ref.shape[0]
    we = we_ref[...].reshape(bb * L, DW)
    mask = (wid_ref[...] != 0).astype(jnp.float32)
    x = jnp.dot(we * mask, wwt_ref[...], preferred_element_type=jnp.float32)
    x = x + bw_ref[...]

    proj_t = jnp.dot(tt_ref[...], wtt_ref[...], preferred_element_type=jnp.float32)
    tid = tid_ref[...]
    iota = lax.broadcasted_iota(jnp.int32, (bb, T), 1)
    oh = ((iota == tid) & (tid != 0)).astype(jnp.float32)
    te = jnp.dot(oh, proj_t, preferred_element_type=jnp.float32) + bt_ref[...]

    x3 = x.reshape(bb, L, H) + te[:, None, :]
    mu = jnp.mean(x3, axis=-1, keepdims=True)
    var = jnp.mean((x3 - mu) * (x3 - mu), axis=-1, keepdims=True)
    y = (x3 - mu) * lax.rsqrt(var + 1e-5)
    out_ref[...] = y * g_ref[...].reshape(1, 1, H) + b_ref[...].reshape(1, 1, H)


def _tc_fused(we3, word_ids, topic_ids, topic_table, w_topic_t, w_word_t,
              b_word, b_topic, gamma, beta, bb=128):
    grid = (B // bb,)
    wid2 = word_ids.reshape(N_ROWS, 1)
    tid2 = topic_ids.reshape(B, 1)
    full2 = lambda shape: pl.BlockSpec(shape, lambda i: (0, 0))
    return pl.pallas_call(
        _tc_body,
        grid=grid,
        in_specs=[
            pl.BlockSpec((bb, L, DW), lambda i: (i, 0, 0)),
            pl.BlockSpec((bb * L, 1), lambda i: (i, 0)),
            pl.BlockSpec((bb, 1), lambda i: (i, 0)),
            full2((T, DT)),
            full2((DT, H)),
            full2((DW, H)),
            full2((1, H)),
            full2((1, H)),
            full2((1, H)),
            full2((1, H)),
        ],
        out_specs=pl.BlockSpec((bb, L, H), lambda i: (i, 0, 0)),
        out_shape=jax.ShapeDtypeStruct((B, L, H), jnp.float32),
        compiler_params=pltpu.CompilerParams(
            dimension_semantics=("arbitrary",)),
    )(we3, wid2, tid2, topic_table, w_topic_t, w_word_t,
      b_word.reshape(1, H), b_topic.reshape(1, H),
      gamma.reshape(1, H), beta.reshape(1, H))


def kernel(word_ids, topic_ids, word_table, topic_table, W_word, b_word,
           W_topic, b_topic, gamma, beta):
    we_flat = _sc_gather(word_ids.reshape(N_ROWS), word_table)
    we3 = we_flat.reshape(B, L, DW)
    return _tc_fused(we3, word_ids, topic_ids, topic_table,
                     W_topic.T, W_word.T, b_word, b_topic, gamma, beta)


# speedup vs baseline: 2.0867x; 1.1252x over previous
import functools

import jax
import jax.numpy as jnp
from jax import lax
from jax.experimental import pallas as pl
from jax.experimental.pallas import tpu as pltpu
from jax.experimental.pallas import tpu_sc as plsc

V, DW, T, DT, H = 100000, 128, 512, 64, 256
B, L = 4096, 50
N_ROWS = B * L

_NC, _NS = 2, 16
_NW = _NC * _NS
_CHUNK = 128
_ROWS_PER_W = N_ROWS // _NW
_CHUNKS_PER_W = _ROWS_PER_W // _CHUNK


def _sc_gather_body(ids_hbm, table_hbm, out_hbm, idx_all, rows, sem):
    wid = lax.axis_index("s") * _NC + lax.axis_index("c")
    chunk_base = wid * _CHUNKS_PER_W
    pltpu.sync_copy(ids_hbm.at[wid], idx_all)

    def chunk(j, carry):
        pltpu.async_copy(table_hbm.at[idx_all.at[j]], rows, sem).wait()
        pltpu.sync_copy(rows, out_hbm.at[pl.ds((chunk_base + j) * _CHUNK, _CHUNK)])
        return carry

    lax.fori_loop(0, _CHUNKS_PER_W, chunk, 0)


def _sc_gather(word_ids_flat, word_table):
    ids2d = word_ids_flat.reshape(_NW, _CHUNKS_PER_W, _CHUNK)
    mesh = plsc.VectorSubcoreMesh(core_axis_name="c", subcore_axis_name="s")
    k = functools.partial(
        pl.kernel,
        mesh=mesh,
        out_type=jax.ShapeDtypeStruct((N_ROWS, DW), jnp.float32),
        scratch_types=[
            pltpu.VMEM((_CHUNKS_PER_W, _CHUNK), jnp.int32),
            pltpu.VMEM((_CHUNK, DW), jnp.float32),
            pltpu.SemaphoreType.DMA,
        ],
    )(_sc_gather_body)
    return k(ids2d, word_table)


def _tc_body(we_ref, wid_ref, tid_ref, tt_ref, wtt_ref, wwt_ref,
             bw_ref, bt_ref, g_ref, b_ref, out_ref, e_ref):
    bb = tid_ref.shape[0]
    rows = bb * L

    @pl.when(pl.program_id(0) == 0)
    def _():
        i0 = lax.broadcasted_iota(jnp.int32, (rows, bb), 0)
        i1 = lax.broadcasted_iota(jnp.int32, (rows, bb), 1)
        e_ref[...] = (i0 // L == i1).astype(jnp.float32)

    wtc = wwt_ref[...]
    wtc = wtc - jnp.mean(wtc, axis=1, keepdims=True)
    p = jnp.dot(tt_ref[...], wtt_ref[...], preferred_element_type=jnp.float32)
    pc = p - jnp.mean(p, axis=1, keepdims=True)
    bc = bw_ref[...] + bt_ref[...]
    bcc = bc - jnp.mean(bc, axis=1, keepdims=True)

    tid = tid_ref[...]
    iota = lax.broadcasted_iota(jnp.int32, (bb, T), 1)
    oh = ((iota == tid) & (tid != 0)).astype(jnp.float32)
    te = jnp.dot(oh, pc, preferred_element_type=jnp.float32) + bcc

    mask = (wid_ref[...] != 0).astype(jnp.float32)
    xc = (jnp.dot(we_ref[...] * mask, wtc, preferred_element_type=jnp.float32)
          + jnp.dot(e_ref[...], te, preferred_element_type=jnp.float32))
    var = jnp.mean(xc * xc, axis=1, keepdims=True)
    y = xc * lax.rsqrt(var + 1e-5)
    out_ref[...] = y * g_ref[...] + b_ref[...]


def _tc_fused(we2, word_ids, topic_ids, topic_table, w_topic_t, w_word_t,
              b_word, b_topic, gamma, beta, bb=128):
    grid = (B // bb,)
    wid2 = word_ids.reshape(N_ROWS, 1)
    tid2 = topic_ids.reshape(B, 1)
    full2 = lambda shape: pl.BlockSpec(shape, lambda i: (0, 0))
    return pl.pallas_call(
        _tc_body,
        grid=grid,
        in_specs=[
            pl.BlockSpec((bb * L, DW), lambda i: (i, 0)),
            pl.BlockSpec((bb * L, 1), lambda i: (i, 0)),
            pl.BlockSpec((bb, 1), lambda i: (i, 0)),
            full2((T, DT)),
            full2((DT, H)),
            full2((DW, H)),
            full2((1, H)),
            full2((1, H)),
            full2((1, H)),
            full2((1, H)),
        ],
        out_specs=pl.BlockSpec((bb * L, H), lambda i: (i, 0)),
        out_shape=jax.ShapeDtypeStruct((N_ROWS, H), jnp.float32),
        scratch_shapes=[pltpu.VMEM((bb * L, bb), jnp.float32)],
        compiler_params=pltpu.CompilerParams(
            dimension_semantics=("arbitrary",)),
    )(we2, wid2, tid2, topic_table, w_topic_t, w_word_t,
      b_word.reshape(1, H), b_topic.reshape(1, H),
      gamma.reshape(1, H), beta.reshape(1, H))


def kernel(word_ids, topic_ids, word_table, topic_table, W_word, b_word,
           W_topic, b_topic, gamma, beta):
    we_flat = _sc_gather(word_ids.reshape(N_ROWS), word_table)
    out2 = _tc_fused(we_flat, word_ids, topic_ids, topic_table,
                     W_topic.T, W_word.T, b_word, b_topic, gamma, beta)
    return out2.reshape(B, L, H)


# speedup vs baseline: 3.0356x; 1.4548x over previous
import functools

import jax
import jax.numpy as jnp
from jax import lax
from jax.experimental import pallas as pl
from jax.experimental.pallas import tpu as pltpu
from jax.experimental.pallas import tpu_sc as plsc

V, DW, T, DT, H = 100000, 128, 512, 64, 256
B, L = 4096, 50
N_ROWS = B * L

_NC, _NS = 2, 16
_NW = _NC * _NS
_CHUNK = 128
_ROWS_PER_W = N_ROWS // _NW
_CHUNKS_PER_W = _ROWS_PER_W // _CHUNK


def _sc_gather_body(ids_hbm, table_hbm, out_hbm, idx_all, rows, sem):
    wid = lax.axis_index("s") * _NC + lax.axis_index("c")
    chunk_base = wid * _CHUNKS_PER_W
    pltpu.sync_copy(ids_hbm.at[wid], idx_all)

    def chunk(j, carry):
        pltpu.async_copy(table_hbm.at[idx_all.at[j]], rows, sem).wait()
        pltpu.sync_copy(rows, out_hbm.at[pl.ds((chunk_base + j) * _CHUNK, _CHUNK)])
        return carry

    lax.fori_loop(0, _CHUNKS_PER_W, chunk, 0)


def _sc_gather(word_ids_flat, word_table):
    ids2d = word_ids_flat.reshape(_NW, _CHUNKS_PER_W, _CHUNK)
    mesh = plsc.VectorSubcoreMesh(core_axis_name="c", subcore_axis_name="s")
    k = functools.partial(
        pl.kernel,
        mesh=mesh,
        out_type=jax.ShapeDtypeStruct((N_ROWS, DW), jnp.float32),
        scratch_types=[
            pltpu.VMEM((_CHUNKS_PER_W, _CHUNK), jnp.int32),
            pltpu.VMEM((_CHUNK, DW), jnp.float32),
            pltpu.SemaphoreType.DMA,
        ],
    )(_sc_gather_body)
    return k(ids2d, word_table)


def _tc_body(we_ref, wid_ref, tid_ref, tt_ref, wtt_ref, wwt_ref,
             bw_ref, bt_ref, g_ref, b_ref, out_ref, e_ref):
    bb = tid_ref.shape[0]
    rows = bb * L

    @pl.when(pl.program_id(0) == 0)
    def _():
        i0 = lax.broadcasted_iota(jnp.int32, (rows, bb), 0)
        i1 = lax.broadcasted_iota(jnp.int32, (rows, bb), 1)
        e_ref[...] = (i0 // L == i1).astype(jnp.float32)

    wtc = wwt_ref[...]
    wtc = wtc - jnp.mean(wtc, axis=1, keepdims=True)
    p = jnp.dot(tt_ref[...], wtt_ref[...], preferred_element_type=jnp.float32)
    pc = p - jnp.mean(p, axis=1, keepdims=True)
    bc = bw_ref[...] + bt_ref[...]
    bcc = bc - jnp.mean(bc, axis=1, keepdims=True)

    tid = tid_ref[...]
    iota = lax.broadcasted_iota(jnp.int32, (bb, T), 1)
    oh = ((iota == tid) & (tid != 0)).astype(jnp.float32)
    te = jnp.dot(oh, pc, preferred_element_type=jnp.float32) + bcc

    mask = (wid_ref[...] != 0).astype(jnp.float32)
    xc = (jnp.dot(we_ref[...] * mask, wtc, preferred_element_type=jnp.float32)
          + jnp.dot(e_ref[...], te, preferred_element_type=jnp.float32))
    var = jnp.mean(xc * xc, axis=1, keepdims=True)
    y = xc * lax.rsqrt(var + 1e-5)
    y = y * g_ref[...] + b_ref[...]
    out_ref[...] = y.reshape(bb, L, H)


def _tc_fused(we2, word_ids, topic_ids, topic_table, w_topic_t, w_word_t,
              b_word, b_topic, gamma, beta, bb=128):
    grid = (B // bb,)
    wid2 = word_ids.reshape(N_ROWS, 1)
    tid2 = topic_ids.reshape(B, 1)
    full2 = lambda shape: pl.BlockSpec(shape, lambda i: (0, 0))
    return pl.pallas_call(
        _tc_body,
        grid=grid,
        in_specs=[
            pl.BlockSpec((bb * L, DW), lambda i: (i, 0)),
            pl.BlockSpec((bb * L, 1), lambda i: (i, 0)),
            pl.BlockSpec((bb, 1), lambda i: (i, 0)),
            full2((T, DT)),
            full2((DT, H)),
            full2((DW, H)),
            full2((1, H)),
            full2((1, H)),
            full2((1, H)),
            full2((1, H)),
        ],
        out_specs=pl.BlockSpec((bb, L, H), lambda i: (i, 0, 0)),
        out_shape=jax.ShapeDtypeStruct((B, L, H), jnp.float32),
        scratch_shapes=[pltpu.VMEM((bb * L, bb), jnp.float32)],
        compiler_params=pltpu.CompilerParams(
            dimension_semantics=("arbitrary",)),
    )(we2, wid2, tid2, topic_table, w_topic_t, w_word_t,
      b_word.reshape(1, H), b_topic.reshape(1, H),
      gamma.reshape(1, H), beta.reshape(1, H))


def kernel(word_ids, topic_ids, word_table, topic_table, W_word, b_word,
           W_topic, b_topic, gamma, beta):
    we_flat = _sc_gather(word_ids.reshape(N_ROWS), word_table)
    return _tc_fused(we_flat, word_ids, topic_ids, topic_table,
                     W_topic.T, W_word.T, b_word, b_topic, gamma, beta)


# speedup vs baseline: 3.0418x; 1.0020x over previous
import functools

import jax
import jax.numpy as jnp
from jax import lax
from jax.experimental import pallas as pl
from jax.experimental.pallas import tpu as pltpu
from jax.experimental.pallas import tpu_sc as plsc

V, DW, T, DT, H = 100000, 128, 512, 64, 256
B, L = 4096, 50
N_ROWS = B * L

_NC, _NS = 2, 16
_NW = _NC * _NS
_CHUNK = 128
_ROWS_PER_W = N_ROWS // _NW
_CHUNKS_PER_W = _ROWS_PER_W // _CHUNK


def _sc_gather_body(chunks_per_w, ids_hbm, table_hbm, out_hbm, idx_all, rows, sem):
    wid = lax.axis_index("s") * _NC + lax.axis_index("c")
    chunk_base = wid * chunks_per_w
    pltpu.sync_copy(ids_hbm.at[wid], idx_all)

    def chunk(j, carry):
        pltpu.async_copy(table_hbm.at[idx_all.at[j]], rows, sem).wait()
        pltpu.sync_copy(rows, out_hbm.at[pl.ds((chunk_base + j) * _CHUNK, _CHUNK)])
        return carry

    lax.fori_loop(0, chunks_per_w, chunk, 0)


def _sc_gather(word_ids_flat, word_table, n_rows, chunks_per_w):
    ids2d = word_ids_flat.reshape(_NW, chunks_per_w, _CHUNK)
    mesh = plsc.VectorSubcoreMesh(core_axis_name="c", subcore_axis_name="s")
    k = functools.partial(
        pl.kernel,
        mesh=mesh,
        out_type=jax.ShapeDtypeStruct((n_rows, DW), jnp.float32),
        scratch_types=[
            pltpu.VMEM((chunks_per_w, _CHUNK), jnp.int32),
            pltpu.VMEM((_CHUNK, DW), jnp.float32),
            pltpu.SemaphoreType.DMA,
        ],
    )(functools.partial(_sc_gather_body, chunks_per_w))
    return k(ids2d, word_table)


def _tc_body(we_ref, wid_ref, tid_ref, tt_ref, wtt_ref, wwt_ref,
             bw_ref, bt_ref, g_ref, b_ref, out_ref, e_ref,
             wtc_ref, pc_ref, bcc_ref):
    bb = tid_ref.shape[0]
    rows = bb * L

    @pl.when(pl.program_id(0) == 0)
    def _():
        i0 = lax.broadcasted_iota(jnp.int32, (rows, bb), 0)
        i1 = lax.broadcasted_iota(jnp.int32, (rows, bb), 1)
        e_ref[...] = (i0 // L == i1).astype(jnp.float32)
        wtc = wwt_ref[...]
        wtc_ref[...] = wtc - jnp.mean(wtc, axis=1, keepdims=True)
        p = jnp.dot(tt_ref[...], wtt_ref[...],
                    preferred_element_type=jnp.float32)
        pc_ref[...] = p - jnp.mean(p, axis=1, keepdims=True)
        bc = bw_ref[...] + bt_ref[...]
        bcc_ref[...] = bc - jnp.mean(bc, axis=1, keepdims=True)

    wtc = wtc_ref[...]

    tid = tid_ref[...]
    iota = lax.broadcasted_iota(jnp.int32, (bb, T), 1)
    oh = ((iota == tid) & (tid != 0)).astype(jnp.float32)
    te = (jnp.dot(oh, pc_ref[...], preferred_element_type=jnp.float32)
          + bcc_ref[...])

    mask = (wid_ref[...] != 0).astype(jnp.float32)
    xc = (jnp.dot(we_ref[...] * mask, wtc, preferred_element_type=jnp.float32)
          + jnp.dot(e_ref[...], te, preferred_element_type=jnp.float32))
    var = jnp.mean(xc * xc, axis=1, keepdims=True)
    y = xc * lax.rsqrt(var + 1e-5)
    y = y * g_ref[...] + b_ref[...]
    out_ref[...] = y.reshape(bb, L, H)


def _tc_fused(we2, word_ids, topic_ids, topic_table, w_topic_t, w_word_t,
              b_word, b_topic, gamma, beta, bb=128):
    grid = (B // bb,)
    wid2 = word_ids.reshape(N_ROWS, 1)
    tid2 = topic_ids.reshape(B, 1)
    full2 = lambda shape: pl.BlockSpec(shape, lambda i: (0, 0))
    return pl.pallas_call(
        _tc_body,
        grid=grid,
        in_specs=[
            pl.BlockSpec((bb * L, DW), lambda i: (i, 0)),
            pl.BlockSpec((bb * L, 1), lambda i: (i, 0)),
            pl.BlockSpec((bb, 1), lambda i: (i, 0)),
            full2((T, DT)),
            full2((DT, H)),
            full2((DW, H)),
            full2((1, H)),
            full2((1, H)),
            full2((1, H)),
            full2((1, H)),
        ],
        out_specs=pl.BlockSpec((bb, L, H), lambda i: (i, 0, 0)),
        out_shape=jax.ShapeDtypeStruct((B, L, H), jnp.float32),
        scratch_shapes=[pltpu.VMEM((bb * L, bb), jnp.float32),
                        pltpu.VMEM((DW, H), jnp.float32),
                        pltpu.VMEM((T, H), jnp.float32),
                        pltpu.VMEM((1, H), jnp.float32)],
        compiler_params=pltpu.CompilerParams(
            dimension_semantics=("arbitrary",)),
    )(we2, wid2, tid2, topic_table, w_topic_t, w_word_t,
      b_word.reshape(1, H), b_topic.reshape(1, H),
      gamma.reshape(1, H), beta.reshape(1, H))


def kernel(word_ids, topic_ids, word_table, topic_table, W_word, b_word,
           W_topic, b_topic, gamma, beta):
    we_flat = _sc_gather(word_ids.reshape(N_ROWS), word_table,
                         N_ROWS, _CHUNKS_PER_W)
    return _tc_fused(we_flat, word_ids, topic_ids, topic_table,
                     W_topic.T, W_word.T, b_word, b_topic, gamma, beta)
